# baseline (speedup 1.0000x reference)
import functools

import jax
import jax.numpy as jnp
from jax import lax
from jax.experimental import pallas as pl
from jax.experimental.pallas import tpu as pltpu
from jax.experimental.pallas import tpu_sc as plsc

N = 10000
E = 320000
D = 128

L = 16
NC = 2
NS = 16

CHUNK = 80
EDGES_PER_TILE = E // (NC * NS)
NCHUNK = EDGES_PER_TILE // CHUNK
NBUF = 3
NDST = 4
ROWS_PER_TILE = N // NS
WROWS = 624


def _sc_spmm_body(dst_hbm, src_hbm, val_hbm, x_hbm, out_hbm,
                  srcb, dstb, valb, rows0, rows1, rows2, acc_sh,
                  gsem0, gsem1, gsem2, hsem0, hsem1, hsem2,
                  ssem0, ssem1, ssem2,
                  isem0, isem1, isem2):
    c = lax.axis_index("c")
    s = lax.axis_index("s")

    tid = c * NS + s

    zv = jnp.zeros((L,), jnp.float32)

    def zrow(r, carry):
        for j in range(D // L):
            rows0[r, pl.ds(j * L, L)] = zv
        return carry

    lax.fori_loop(0, CHUNK, zrow, 0)
    for k in range(ROWS_PER_TILE // CHUNK):
        pltpu.sync_copy(rows0,
                        acc_sh.at[pl.ds(s * ROWS_PER_TILE + k * CHUNK, CHUNK)])
    ztail = ROWS_PER_TILE % CHUNK
    if ztail:
        pltpu.sync_copy(
            rows0.at[pl.ds(0, ztail)],
            acc_sh.at[pl.ds(s * ROWS_PER_TILE + ROWS_PER_TILE - ztail, ztail)])
    plsc.subcore_barrier()

    rows = (rows0, rows1, rows2)
    gsems = (gsem0, gsem1, gsem2)
    hsems = (hsem0, hsem1, hsem2)
    ssems = (ssem0, ssem1, ssem2)
    isems = (isem0, isem1, isem2)
    HC = CHUNK // 2

    ebase = tid * EDGES_PER_TILE

    def start_idx(i, b):
        sl = pl.ds(ebase + i * CHUNK, CHUNK)
        pltpu.async_copy(src_hbm.at[sl], srcb.at[b], isems[b])
        pltpu.async_copy(dst_hbm.at[sl], dstb.at[i % NDST], isems[b])
        pltpu.async_copy(val_hbm.at[sl], valb.at[b], isems[b])

    def wait_idx(b):
        sl = pl.ds(0, CHUNK)
        pltpu.make_async_copy(src_hbm.at[sl], srcb.at[b], isems[b]).wait()
        pltpu.make_async_copy(dst_hbm.at[sl], dstb.at[0], isems[b]).wait()
        pltpu.make_async_copy(val_hbm.at[sl], valb.at[b], isems[b]).wait()

    def start_gather(b):
        pltpu.async_copy(x_hbm.at[srcb.at[b, pl.ds(0, HC)]],
                         rows[b].at[pl.ds(0, HC)], gsems[b])
        pltpu.async_copy(x_hbm.at[srcb.at[b, pl.ds(HC, HC)]],
                         rows[b].at[pl.ds(HC, HC)], hsems[b])

    def wait_gather(b):
        pltpu.make_async_copy(x_hbm.at[srcb.at[0, pl.ds(0, HC)]],
                              rows[b].at[pl.ds(0, HC)], gsems[b]).wait()
        pltpu.make_async_copy(x_hbm.at[srcb.at[0, pl.ds(0, HC)]],
                              rows[b].at[pl.ds(HC, HC)], hsems[b]).wait()

    def start_scatter(b, k):
        pltpu.async_copy(rows[b], acc_sh.at[dstb.at[k % NDST]], ssems[b],
                         add=True)

    def wait_scatter(b):
        pltpu.make_async_copy(rows[b], acc_sh.at[dstb.at[0]], ssems[b]).wait()

    for b in range(NBUF):
        start_idx(b, b)
    wait_idx(0)
    start_gather(0)
    wait_idx(1)
    start_gather(1)

    def process(k, rb, first):
        wait_gather(rb)

        @plsc.parallel_loop(0, CHUNK // L, unroll=2)
        def scale_grp(g):
            vv16 = valb[rb, pl.ds(g * L, L)]
            for t in range(L):
                vv = vv16[t]
                e = g * L + t
                for j in range(D // L):
                    sl = pl.ds(j * L, L)
                    rows[rb][e, sl] = rows[rb][e, sl] * vv
        start_scatter(rb, k)
        pb = (rb + NBUF - 1) % NBUF
        if not first:
            wait_scatter(pb)

        @pl.when(k + NBUF < NCHUNK)
        def _():
            start_idx(k + NBUF, rb)

        @pl.when(k + 2 < NCHUNK)
        def _():
            wait_idx(pb)
            start_gather(pb)

    process(0, 0, True)
    process(1, 1, False)

    def outer(io, carry):
        k = io * NBUF + 2
        for p in range(NBUF):
            process(k + p, (2 + p) % NBUF, False)
        return carry

    lax.fori_loop(0, (NCHUNK - 2) // NBUF, outer, 0)
    wait_scatter((NCHUNK - 1) % NBUF)
    plsc.subcore_barrier()

    w0 = s * WROWS
    pltpu.sync_copy(acc_sh.at[pl.ds(w0, WROWS)],
                    out_hbm.at[c, pl.ds(w0, WROWS)])

    @pl.when(s == 0)
    def _write_tail():
        pltpu.sync_copy(acc_sh.at[pl.ds(NS * WROWS, N - NS * WROWS)],
                        out_hbm.at[c, pl.ds(NS * WROWS, N - NS * WROWS)])


_sc_spmm = functools.partial(
    pl.kernel,
    out_type=jax.ShapeDtypeStruct((NC, N, D), jnp.float32),
    mesh=plsc.VectorSubcoreMesh(core_axis_name="c", subcore_axis_name="s"),
    scratch_types=[
        pltpu.VMEM((NBUF, CHUNK), jnp.int32),
        pltpu.VMEM((NDST, CHUNK), jnp.int32),
        pltpu.VMEM((NBUF, CHUNK), jnp.float32),
        pltpu.VMEM((CHUNK, D), jnp.float32),
        pltpu.VMEM((CHUNK, D), jnp.float32),
        pltpu.VMEM((CHUNK, D), jnp.float32),
        pltpu.VMEM_SHARED((N, D), jnp.float32),
        pltpu.SemaphoreType.DMA,
        pltpu.SemaphoreType.DMA,
        pltpu.SemaphoreType.DMA,
        pltpu.SemaphoreType.DMA,
        pltpu.SemaphoreType.DMA,
        pltpu.SemaphoreType.DMA,
        pltpu.SemaphoreType.DMA,
        pltpu.SemaphoreType.DMA,
        pltpu.SemaphoreType.DMA,
        pltpu.SemaphoreType.DMA,
        pltpu.SemaphoreType.DMA,
        pltpu.SemaphoreType.DMA,
    ],
)(_sc_spmm_body)


BLK = 1000


def _linear_body(p_ref, w_ref, b_ref, o_ref):
    acc = p_ref[0] + p_ref[1]
    o_ref[...] = lax.dot_general(
        acc, w_ref[...], (((1,), (1,)), ((), ())),
        preferred_element_type=jnp.float32) + b_ref[...]


_linear = pl.pallas_call(
    _linear_body,
    grid=(N // BLK,),
    in_specs=[
        pl.BlockSpec((NC, BLK, D), lambda i: (0, i, 0)),
        pl.BlockSpec((D, D), lambda i: (0, 0)),
        pl.BlockSpec((1, D), lambda i: (0, 0)),
    ],
    out_specs=pl.BlockSpec((BLK, D), lambda i: (i, 0)),
    out_shape=jax.ShapeDtypeStruct((N, D), jnp.float32),
)


def kernel(A_indices, A_values, X, W, b):
    dst = A_indices[0].astype(jnp.int32)
    src = A_indices[1].astype(jnp.int32)
    vals = A_values
    partials = _sc_spmm(dst, src, vals, X)
    return _linear(partials, W, b.reshape(1, D))


# speedup vs baseline: 1.0093x; 1.0093x over previous
import functools

import jax
import jax.numpy as jnp
from jax import lax
from jax.experimental import pallas as pl
from jax.experimental.pallas import tpu as pltpu
from jax.experimental.pallas import tpu_sc as plsc

N = 10000
E = 320000
D = 128

L = 16
NC = 2
NS = 16

CHUNK = 80
EDGES_PER_TILE = E // (NC * NS)
NCHUNK = EDGES_PER_TILE // CHUNK
NBUF = 3
NDST = 4
ROWS_PER_TILE = N // NS
WROWS = 624


def _sc_spmm_body(dst_hbm, src_hbm, val_hbm, x_hbm, out_hbm,
                  srcb, dstb, valb, rows0, rows1, rows2, acc_sh,
                  gsem0, gsem1, gsem2, ssem0, ssem1, ssem2,
                  isem0, isem1, isem2):
    c = lax.axis_index("c")
    s = lax.axis_index("s")

    tid = c * NS + s

    rows = (rows0, rows1, rows2)
    gsems = (gsem0, gsem1, gsem2)
    ssems = (ssem0, ssem1, ssem2)
    isems = (isem0, isem1, isem2)

    ebase = tid * EDGES_PER_TILE

    def start_idx(i, b):
        sl = pl.ds(ebase + i * CHUNK, CHUNK)
        pltpu.async_copy(src_hbm.at[sl], srcb.at[b], isems[b])
        pltpu.async_copy(dst_hbm.at[sl], dstb.at[i % NDST], isems[b])
        pltpu.async_copy(val_hbm.at[sl], valb.at[b], isems[b])

    def wait_idx(b):
        sl = pl.ds(0, CHUNK)
        pltpu.make_async_copy(src_hbm.at[sl], srcb.at[b], isems[b]).wait()
        pltpu.make_async_copy(dst_hbm.at[sl], dstb.at[0], isems[b]).wait()
        pltpu.make_async_copy(val_hbm.at[sl], valb.at[b], isems[b]).wait()

    def start_gather(b):
        pltpu.async_copy(x_hbm.at[srcb.at[b]], rows[b], gsems[b])

    def wait_gather(b):
        pltpu.make_async_copy(x_hbm.at[srcb.at[0]], rows[b], gsems[b]).wait()

    def start_scatter(b, k):
        pltpu.async_copy(rows[b], acc_sh.at[dstb.at[k % NDST]], ssems[b],
                         add=True)

    def wait_scatter(b):
        pltpu.make_async_copy(rows[b], acc_sh.at[dstb.at[0]], ssems[b]).wait()

    for b in range(NBUF):
        start_idx(b, b)
    wait_idx(0)
    start_gather(0)
    wait_idx(1)
    start_gather(1)

    zv = jnp.zeros((L,), jnp.float32)

    def zrow(r, carry):
        for j in range(D // L):
            rows2[r, pl.ds(j * L, L)] = zv
        return carry

    lax.fori_loop(0, CHUNK, zrow, 0)
    for k in range(ROWS_PER_TILE // CHUNK):
        pltpu.sync_copy(rows2,
                        acc_sh.at[pl.ds(s * ROWS_PER_TILE + k * CHUNK, CHUNK)])
    ztail = ROWS_PER_TILE % CHUNK
    if ztail:
        pltpu.sync_copy(
            rows2.at[pl.ds(0, ztail)],
            acc_sh.at[pl.ds(s * ROWS_PER_TILE + ROWS_PER_TILE - ztail, ztail)])
    plsc.subcore_barrier()

    def process(k, rb, first):
        wait_gather(rb)

        @plsc.parallel_loop(0, CHUNK // L, unroll=2)
        def scale_grp(g):
            vv16 = valb[rb, pl.ds(g * L, L)]
            for t in range(L):
                vv = vv16[t]
                e = g * L + t
                for j in range(D // L):
                    sl = pl.ds(j * L, L)
                    rows[rb][e, sl] = rows[rb][e, sl] * vv
        start_scatter(rb, k)
        pb = (rb + NBUF - 1) % NBUF
        if not first:
            wait_scatter(pb)

        @pl.when(k + NBUF < NCHUNK)
        def _():
            start_idx(k + NBUF, rb)

        @pl.when(k + 2 < NCHUNK)
        def _():
            wait_idx(pb)
            start_gather(pb)

    process(0, 0, True)
    process(1, 1, False)

    def outer(io, carry):
        k = io * NBUF + 2
        for p in range(NBUF):
            process(k + p, (2 + p) % NBUF, False)
        return carry

    lax.fori_loop(0, (NCHUNK - 2) // NBUF, outer, 0)
    wait_scatter((NCHUNK - 1) % NBUF)
    plsc.subcore_barrier()

    w0 = s * WROWS
    pltpu.sync_copy(acc_sh.at[pl.ds(w0, WROWS)],
                    out_hbm.at[c, pl.ds(w0, WROWS)])

    @pl.when(s == 0)
    def _write_tail():
        pltpu.sync_copy(acc_sh.at[pl.ds(NS * WROWS, N - NS * WROWS)],
                        out_hbm.at[c, pl.ds(NS * WROWS, N - NS * WROWS)])


_sc_spmm = functools.partial(
    pl.kernel,
    out_type=jax.ShapeDtypeStruct((NC, N, D), jnp.float32),
    mesh=plsc.VectorSubcoreMesh(core_axis_name="c", subcore_axis_name="s"),
    scratch_types=[
        pltpu.VMEM((NBUF, CHUNK), jnp.int32),
        pltpu.VMEM((NDST, CHUNK), jnp.int32),
        pltpu.VMEM((NBUF, CHUNK), jnp.float32),
        pltpu.VMEM((CHUNK, D), jnp.float32),
        pltpu.VMEM((CHUNK, D), jnp.float32),
        pltpu.VMEM((CHUNK, D), jnp.float32),
        pltpu.VMEM_SHARED((N, D), jnp.float32),
        pltpu.SemaphoreType.DMA,
        pltpu.SemaphoreType.DMA,
        pltpu.SemaphoreType.DMA,
        pltpu.SemaphoreType.DMA,
        pltpu.SemaphoreType.DMA,
        pltpu.SemaphoreType.DMA,
        pltpu.SemaphoreType.DMA,
        pltpu.SemaphoreType.DMA,
        pltpu.SemaphoreType.DMA,
    ],
)(_sc_spmm_body)


BLK = 1000


def _linear_body(p_ref, w_ref, b_ref, o_ref):
    acc = p_ref[0] + p_ref[1]
    o_ref[...] = lax.dot_general(
        acc, w_ref[...], (((1,), (1,)), ((), ())),
        preferred_element_type=jnp.float32) + b_ref[...]


_linear = pl.pallas_call(
    _linear_body,
    grid=(N // BLK,),
    in_specs=[
        pl.BlockSpec((NC, BLK, D), lambda i: (0, i, 0)),
        pl.BlockSpec((D, D), lambda i: (0, 0)),
        pl.BlockSpec((1, D), lambda i: (0, 0)),
    ],
    out_specs=pl.BlockSpec((BLK, D), lambda i: (i, 0)),
    out_shape=jax.ShapeDtypeStruct((N, D), jnp.float32),
)


def kernel(A_indices, A_values, X, W, b):
    dst = A_indices[0].astype(jnp.int32)
    src = A_indices[1].astype(jnp.int32)
    vals = A_values
    partials = _sc_spmm(dst, src, vals, X)
    return _linear(partials, W, b.reshape(1, D))
